# baseline (speedup 1.0000x reference)
import jax
import jax.numpy as jnp
from jax import lax
from jax.experimental import pallas as pl
from jax.experimental.pallas import tpu as pltpu
from jax.experimental.pallas import tpu_sc as plsc

B = 16384
DIM = 32
NEG = 20
LANES = 16

VOCAB = 1000000
V4 = VOCAB // 4
SUP = 128

NC = 2
NS = 16
NW = NC * NS

B_PER_W = B // NW
CB = 32
NCH = B_PER_W // CB
NEG_CB = CB * NEG
NEG_STREAMS = NEG_CB // 128


def _w2v_body(insup_hbm, inoff_hbm, ctxsup_hbm, ctxoff_hbm,
              negsup_hbm, negoff_hbm, enc_hbm, dec_hbm,
              pos_out, neg_out,
              widx, woff, cidx, coff, nidx, noff,
              wrows, crows, nrows, posbuf, negbuf, sem):
  wid = lax.axis_index("s") * NC + lax.axis_index("c")

  lane = lax.iota(jnp.int32, LANES)
  _dn = lax.GatherDimensionNumbers(
      offset_dims=(), collapsed_slice_dims=(0,), start_index_map=(0,))
  _pidx = {s: jnp.bitwise_xor(lane, s)[:, None] for s in (8, 4, 2, 1)}
  _pmask = {s: (lane & s) == 0 for s in (8, 4, 2, 1)}

  def _perm(v, s):
    return lax.gather(v, _pidx[s], _dn, (1,),
                      mode=lax.GatherScatterMode.PROMISE_IN_BOUNDS)

  def _combine(a, b, s):
    m = _pmask[s]
    return jnp.where(m, a, _perm(b, s)) + jnp.where(m, _perm(a, s), b)

  def chunk_body(c, _):
    base = pl.multiple_of((wid * NCH + c) * CB, CB)
    nbase = pl.multiple_of(base * NEG, NEG_CB)
    pltpu.sync_copy(insup_hbm.at[pl.ds(base, CB)], widx)
    pltpu.sync_copy(inoff_hbm.at[pl.ds(base, CB)], woff.at[pl.ds(0, CB)])
    pltpu.sync_copy(ctxsup_hbm.at[pl.ds(base, CB)], cidx)
    pltpu.sync_copy(ctxoff_hbm.at[pl.ds(base, CB)], coff.at[pl.ds(0, CB)])
    pltpu.sync_copy(negsup_hbm.at[pl.ds(nbase, NEG_CB)], nidx)
    pltpu.sync_copy(negoff_hbm.at[pl.ds(nbase, NEG_CB)],
                    noff.at[pl.ds(0, NEG_CB)])

    handles = [
        pltpu.async_copy(enc_hbm.at[widx], wrows, sem),
        pltpu.async_copy(dec_hbm.at[cidx], crows, sem),
    ]
    for j in range(NEG_STREAMS):
      handles.append(
          pltpu.async_copy(dec_hbm.at[nidx.at[pl.ds(j * 128, 128)]],
                           nrows.at[pl.ds(j * 128, 128)], sem))
    for h in handles:
      h.wait()

    def b_body(i, _):
      ow = woff[pl.ds(i, LANES)][0]
      w0 = wrows[i, pl.ds(ow, LANES)]
      w1 = wrows[i, pl.ds(ow + LANES, LANES)]
      oc = coff[pl.ds(i, LANES)][0]
      c0 = crows[i, pl.ds(oc, LANES)]
      c1 = crows[i, pl.ds(oc + LANES, LANES)]
      p = w0 * c0 + w1 * c1
      for s in (8, 4, 2, 1):
        p = p + _perm(p, s)
      posbuf[pl.ds(i, LANES)] = p
      r0 = i * NEG
      onv0 = noff[pl.ds(r0, LANES)]
      onv1 = noff[pl.ds(r0 + LANES, LANES)]
      h = []
      for k in range(NEG):
        on = onv0[k] if k < LANES else onv1[k - LANES]
        n0 = nrows[r0 + k, pl.ds(on, LANES)]
        n1 = nrows[r0 + k, pl.ds(on + LANES, LANES)]
        h.append(w0 * n0 + w1 * n1)
      vs = h[:LANES]
      for s in (8, 4, 2, 1):
        half = len(vs) // 2
        vs = [_combine(vs[j], vs[j + half], s) for j in range(half)]
      negbuf[pl.ds(r0, LANES)] = vs[0]
      rs = []
      for k in range(LANES, NEG):
        t = h[k]
        t = t + _perm(t, 8)
        t = t + _perm(t, 4)
        rs.append(t)
      rs = [_combine(rs[j], rs[j + 2], 2) for j in range(2)]
      rs = [_combine(rs[0], rs[1], 1)]
      negbuf[pl.ds(r0 + LANES, LANES)] = rs[0]
      return 0

    lax.fori_loop(0, CB, b_body, 0)

    pltpu.sync_copy(posbuf.at[pl.ds(0, CB)], pos_out.at[pl.ds(base, CB)])
    pltpu.sync_copy(negbuf.at[pl.ds(0, NEG_CB)],
                    neg_out.at[pl.ds(nbase, NEG_CB)])
    return 0

  lax.fori_loop(0, NCH, chunk_body, 0)


@jax.jit
def _w2v_call(in_sup, in_off, ctx_sup, ctx_off, neg_sup, neg_off, enc4, dec4):
  mesh = plsc.VectorSubcoreMesh(core_axis_name="c", subcore_axis_name="s")
  kern = pl.kernel(
      _w2v_body,
      out_type=(
          jax.ShapeDtypeStruct((B,), jnp.float32),
          jax.ShapeDtypeStruct((B * NEG,), jnp.float32),
      ),
      mesh=mesh,
      scratch_types=[
          pltpu.VMEM((CB,), jnp.int32),
          pltpu.VMEM((CB + LANES,), jnp.int32),
          pltpu.VMEM((CB,), jnp.int32),
          pltpu.VMEM((CB + LANES,), jnp.int32),
          pltpu.VMEM((NEG_CB,), jnp.int32),
          pltpu.VMEM((NEG_CB + LANES,), jnp.int32),
          pltpu.VMEM((CB, SUP), jnp.float32),
          pltpu.VMEM((CB, SUP), jnp.float32),
          pltpu.VMEM((NEG_CB, SUP), jnp.float32),
          pltpu.VMEM((CB + LANES,), jnp.float32),
          pltpu.VMEM((NEG_CB + LANES,), jnp.float32),
          pltpu.SemaphoreType.DMA,
      ],
      compiler_params=pltpu.CompilerParams(use_tc_tiling_on_sc=True),
  )
  return kern(in_sup, in_off, ctx_sup, ctx_off, neg_sup, neg_off, enc4, dec4)


def kernel(input_tokens, ctx_tokens, neg_tokens, encoder_weight, decoder_weight):
  in_flat = input_tokens.reshape(B).astype(jnp.int32)
  ctx_flat = ctx_tokens.reshape(B).astype(jnp.int32)
  neg_flat = neg_tokens.reshape(B * NEG).astype(jnp.int32)
  in_sup, in_off = in_flat >> 2, (in_flat & 3) * DIM
  ctx_sup, ctx_off = ctx_flat >> 2, (ctx_flat & 3) * DIM
  neg_sup, neg_off = neg_flat >> 2, (neg_flat & 3) * DIM
  enc4 = encoder_weight.reshape(V4, SUP)
  dec4 = decoder_weight.reshape(V4, SUP)
  pos, neg = _w2v_call(in_sup, in_off, ctx_sup, ctx_off, neg_sup, neg_off,
                       enc4, dec4)
  return pos.reshape(B, 1, 1), neg.reshape(B, 1, NEG)


# speedup vs baseline: 1.8554x; 1.8554x over previous
import jax
import jax.numpy as jnp
from jax import lax
from jax.experimental import pallas as pl
from jax.experimental.pallas import tpu as pltpu
from jax.experimental.pallas import tpu_sc as plsc

B = 16384
DIM = 32
NEG = 20
LANES = 16

VOCAB = 1000000
V4 = VOCAB // 4
SUP = 128

NC = 2
NS = 16
NW = NC * NS

NCHT = (VOCAB + 127) // 128
V4P = NCHT * 32
TCPW = (NCHT + NW - 1) // NW

B_PER_W = B // NW
CB = 32
NCH = B_PER_W // CB
NEG_CB = CB * NEG
NEG_STREAMS = NEG_CB // 128


def _transpose_body(enc_t, dec_t, enc4, dec4, tin, tout, insem, outsem):
  wid = lax.axis_index("s") * NC + lax.axis_index("c")

  lane = lax.iota(jnp.int32, LANES)
  _dn = lax.GatherDimensionNumbers(
      offset_dims=(), collapsed_slice_dims=(0,), start_index_map=(0,))
  _pidx = {s: jnp.bitwise_xor(lane, s)[:, None] for s in (8, 4, 2, 1)}
  _pmask = {s: (lane & s) == 0 for s in (8, 4, 2, 1)}

  def _perm(v, s):
    return lax.gather(v, _pidx[s], _dn, (1,),
                      mode=lax.GatherScatterMode.PROMISE_IN_BOUNDS)

  def do_table(src, dst):
    nloc = jnp.where(wid < NCHT - (TCPW - 1) * NW, TCPW, TCPW - 1)

    def fire_in(c, p):
      col0 = pl.multiple_of((wid + NW * c) * 128, 128)
      pltpu.async_copy(src.at[:, pl.ds(col0, 128)], tin.at[p], insem)

    def fire_out(c, p):
      r0 = pl.multiple_of((wid + NW * c) * 32, 32)
      pltpu.async_copy(tout.at[p], dst.at[pl.ds(r0, 32), :], outsem)

    fire_in(0, 0)

    def body(c, _):
      p = lax.rem(c, 2)

      @pl.when(c + 1 < nloc)
      def _():
        fire_in(c + 1, 1 - p)

      pltpu.make_async_copy(src.at[:, pl.ds(0, 128)], tin.at[0], insem).wait()
      @pl.when(c >= 2)
      def _():
        pltpu.make_async_copy(tout.at[0], dst.at[pl.ds(0, 32), :],
                              outsem).wait()

      for db in range(2):
        for cb in range(8):
          vs = [tin[p, LANES * db + i, pl.ds(LANES * cb, LANES)]
                for i in range(LANES)]
          for s in (8, 4, 2, 1):
            m = _pmask[s]
            nv = list(vs)
            for i in range(LANES):
              if i & s:
                continue
              j = i ^ s
              a, b = vs[i], vs[j]
              nv[i] = jnp.where(m, a, _perm(b, s))
              nv[j] = jnp.where(m, _perm(a, s), b)
            vs = nv
          for j in range(LANES):
            tout[p, 4 * cb + (j >> 2),
                 pl.ds((j & 3) * DIM + LANES * db, LANES)] = vs[j]

      fire_out(c, p)
      return 0

    lax.fori_loop(0, nloc, body, 0)
    @pl.when(nloc >= 2)
    def _():
      pltpu.make_async_copy(tout.at[0], dst.at[pl.ds(0, 32), :],
                            outsem).wait()
    pltpu.make_async_copy(tout.at[0], dst.at[pl.ds(0, 32), :], outsem).wait()

  do_table(enc_t, enc4)
  do_table(dec_t, dec4)


def _w2v_body(insup_hbm, inoff_hbm, ctxsup_hbm, ctxoff_hbm,
              negsup_hbm, negoff_hbm, enc_hbm, dec_hbm,
              pos_out, neg_out,
              widx, woff, cidx, coff, nidx, noff,
              wrows, crows, nrows, posbuf, negbuf, sem):
  wid = lax.axis_index("s") * NC + lax.axis_index("c")

  lane = lax.iota(jnp.int32, LANES)
  _dn = lax.GatherDimensionNumbers(
      offset_dims=(), collapsed_slice_dims=(0,), start_index_map=(0,))
  _pidx = {s: jnp.bitwise_xor(lane, s)[:, None] for s in (8, 4, 2, 1)}
  _pmask = {s: (lane & s) == 0 for s in (8, 4, 2, 1)}

  def _perm(v, s):
    return lax.gather(v, _pidx[s], _dn, (1,),
                      mode=lax.GatherScatterMode.PROMISE_IN_BOUNDS)

  def _combine(a, b, s):
    m = _pmask[s]
    return jnp.where(m, a, _perm(b, s)) + jnp.where(m, _perm(a, s), b)

  def chunk_body(c, _):
    base = pl.multiple_of((wid * NCH + c) * CB, CB)
    nbase = pl.multiple_of(base * NEG, NEG_CB)
    pltpu.sync_copy(insup_hbm.at[pl.ds(base, CB)], widx)
    pltpu.sync_copy(inoff_hbm.at[pl.ds(base, CB)], woff.at[pl.ds(0, CB)])
    pltpu.sync_copy(ctxsup_hbm.at[pl.ds(base, CB)], cidx)
    pltpu.sync_copy(ctxoff_hbm.at[pl.ds(base, CB)], coff.at[pl.ds(0, CB)])
    pltpu.sync_copy(negsup_hbm.at[pl.ds(nbase, NEG_CB)], nidx)
    pltpu.sync_copy(negoff_hbm.at[pl.ds(nbase, NEG_CB)],
                    noff.at[pl.ds(0, NEG_CB)])

    handles = [
        pltpu.async_copy(enc_hbm.at[widx], wrows, sem),
        pltpu.async_copy(dec_hbm.at[cidx], crows, sem),
    ]
    for j in range(NEG_STREAMS):
      handles.append(
          pltpu.async_copy(dec_hbm.at[nidx.at[pl.ds(j * 128, 128)]],
                           nrows.at[pl.ds(j * 128, 128)], sem))
    for h in handles:
      h.wait()

    def b_body(i, _):
      ow = woff[pl.ds(i, LANES)][0]
      w0 = wrows[i, pl.ds(ow, LANES)]
      w1 = wrows[i, pl.ds(ow + LANES, LANES)]
      oc = coff[pl.ds(i, LANES)][0]
      c0 = crows[i, pl.ds(oc, LANES)]
      c1 = crows[i, pl.ds(oc + LANES, LANES)]
      p = w0 * c0 + w1 * c1
      for s in (8, 4, 2, 1):
        p = p + _perm(p, s)
      posbuf[pl.ds(i, LANES)] = p
      r0 = i * NEG
      onv0 = noff[pl.ds(r0, LANES)]
      onv1 = noff[pl.ds(r0 + LANES, LANES)]
      h = []
      for k in range(NEG):
        on = onv0[k] if k < LANES else onv1[k - LANES]
        n0 = nrows[r0 + k, pl.ds(on, LANES)]
        n1 = nrows[r0 + k, pl.ds(on + LANES, LANES)]
        h.append(w0 * n0 + w1 * n1)
      vs = h[:LANES]
      for s in (8, 4, 2, 1):
        half = len(vs) // 2
        vs = [_combine(vs[j], vs[j + half], s) for j in range(half)]
      negbuf[pl.ds(r0, LANES)] = vs[0]
      rs = []
      for k in range(LANES, NEG):
        t = h[k]
        t = t + _perm(t, 8)
        t = t + _perm(t, 4)
        rs.append(t)
      rs = [_combine(rs[j], rs[j + 2], 2) for j in range(2)]
      rs = [_combine(rs[0], rs[1], 1)]
      negbuf[pl.ds(r0 + LANES, LANES)] = rs[0]
      return 0

    lax.fori_loop(0, CB, b_body, 0)

    pltpu.sync_copy(posbuf.at[pl.ds(0, CB)], pos_out.at[pl.ds(base, CB)])
    pltpu.sync_copy(negbuf.at[pl.ds(0, NEG_CB)],
                    neg_out.at[pl.ds(nbase, NEG_CB)])
    return 0

  lax.fori_loop(0, NCH, chunk_body, 0)


@jax.jit
def _w2v_call(in_sup, in_off, ctx_sup, ctx_off, neg_sup, neg_off,
              enc_t, dec_t):
  mesh = plsc.VectorSubcoreMesh(core_axis_name="c", subcore_axis_name="s")
  conv = pl.kernel(
      _transpose_body,
      out_type=(
          jax.ShapeDtypeStruct((V4P, SUP), jnp.float32),
          jax.ShapeDtypeStruct((V4P, SUP), jnp.float32),
      ),
      mesh=mesh,
      scratch_types=[
          pltpu.VMEM((2, DIM, 128), jnp.float32),
          pltpu.VMEM((2, DIM, 128), jnp.float32),
          pltpu.SemaphoreType.DMA,
          pltpu.SemaphoreType.DMA,
      ],
      compiler_params=pltpu.CompilerParams(use_tc_tiling_on_sc=True),
  )
  enc4, dec4 = conv(enc_t, dec_t)
  kern = pl.kernel(
      _w2v_body,
      out_type=(
          jax.ShapeDtypeStruct((B,), jnp.float32),
          jax.ShapeDtypeStruct((B * NEG,), jnp.float32),
      ),
      mesh=mesh,
      scratch_types=[
          pltpu.VMEM((CB,), jnp.int32),
          pltpu.VMEM((CB + LANES,), jnp.int32),
          pltpu.VMEM((CB,), jnp.int32),
          pltpu.VMEM((CB + LANES,), jnp.int32),
          pltpu.VMEM((NEG_CB,), jnp.int32),
          pltpu.VMEM((NEG_CB + LANES,), jnp.int32),
          pltpu.VMEM((CB, SUP), jnp.float32),
          pltpu.VMEM((CB, SUP), jnp.float32),
          pltpu.VMEM((NEG_CB, SUP), jnp.float32),
          pltpu.VMEM((CB + LANES,), jnp.float32),
          pltpu.VMEM((NEG_CB + LANES,), jnp.float32),
          pltpu.SemaphoreType.DMA,
      ],
      compiler_params=pltpu.CompilerParams(use_tc_tiling_on_sc=True),
  )
  return kern(in_sup, in_off, ctx_sup, ctx_off, neg_sup, neg_off, enc4, dec4)


def kernel(input_tokens, ctx_tokens, neg_tokens, encoder_weight, decoder_weight):
  in_flat = input_tokens.reshape(B).astype(jnp.int32)
  ctx_flat = ctx_tokens.reshape(B).astype(jnp.int32)
  neg_flat = neg_tokens.reshape(B * NEG).astype(jnp.int32)
  in_sup, in_off = in_flat >> 2, (in_flat & 3) * DIM
  ctx_sup, ctx_off = ctx_flat >> 2, (ctx_flat & 3) * DIM
  neg_sup, neg_off = neg_flat >> 2, (neg_flat & 3) * DIM
  pos, neg = _w2v_call(in_sup, in_off, ctx_sup, ctx_off, neg_sup, neg_off,
                       encoder_weight.T, decoder_weight.T)
  return pos.reshape(B, 1, 1), neg.reshape(B, 1, NEG)


# speedup vs baseline: 2.1970x; 1.1841x over previous
import jax
import jax.numpy as jnp
from jax import lax
from jax.experimental import pallas as pl
from jax.experimental.pallas import tpu as pltpu
from jax.experimental.pallas import tpu_sc as plsc

B = 16384
DIM = 32
NEG = 20
LANES = 16

VOCAB = 1000000

NC = 2
NS = 16
NW = NC * NS

NCHT = (VOCAB + 127) // 128
V4P = NCHT * 32
VP = V4P * 4
TCPW = (NCHT + NW - 1) // NW

B_PER_W = B // NW
CB = 128
NCH = B_PER_W // CB
NEG_CB = CB * NEG
NEG_STREAMS = NEG_CB // 128


def _lane_helpers():
  lane = lax.iota(jnp.int32, LANES)
  dn = lax.GatherDimensionNumbers(
      offset_dims=(), collapsed_slice_dims=(0,), start_index_map=(0,))
  pidx = {s: jnp.bitwise_xor(lane, s)[:, None] for s in (8, 4, 2, 1)}
  pmask = {s: (lane & s) == 0 for s in (8, 4, 2, 1)}

  def perm(v, s):
    return lax.gather(v, pidx[s], dn, (1,),
                      mode=lax.GatherScatterMode.PROMISE_IN_BOUNDS)

  return perm, pmask


def _transpose_body(enc_t, dec_t, enc4, dec4, tin, tout, insem, outsem):
  wid = lax.axis_index("s") * NC + lax.axis_index("c")
  perm, pmask = _lane_helpers()

  def do_table(src, dst):
    nloc = jnp.where(wid < NCHT - (TCPW - 1) * NW, TCPW, TCPW - 1)

    def fire_in(c, p):
      col0 = pl.multiple_of((wid + NW * c) * 128, 128)
      pltpu.async_copy(src.at[:, pl.ds(col0, 128)], tin.at[p], insem)

    def fire_out(c, p):
      r0 = pl.multiple_of((wid + NW * c) * 32, 32)
      pltpu.async_copy(tout.at[p], dst.at[pl.ds(r0, 32), :], outsem)

    fire_in(0, 0)

    def body(c, _):
      p = c & 1

      @pl.when(c + 1 < nloc)
      def _():
        fire_in(c + 1, 1 - p)

      pltpu.make_async_copy(src.at[:, pl.ds(0, 128)], tin.at[0], insem).wait()
      @pl.when(c >= 2)
      def _():
        pltpu.make_async_copy(tout.at[0], dst.at[pl.ds(0, 32), :],
                              outsem).wait()

      for db in range(2):
        for cb in range(8):
          vs = [tin[p, LANES * db + i, pl.ds(LANES * cb, LANES)]
                for i in range(LANES)]
          for s in (8, 4, 2, 1):
            m = pmask[s]
            nv = list(vs)
            for i in range(LANES):
              if i & s:
                continue
              j = i ^ s
              a, b = vs[i], vs[j]
              pm = perm(jnp.where(m, b, a), s)
              nv[i] = jnp.where(m, a, pm)
              nv[j] = jnp.where(m, pm, b)
            vs = nv
          for j in range(LANES):
            tout[p, 4 * cb + (j >> 2),
                 pl.ds((j & 3) * DIM + LANES * db, LANES)] = vs[j]

      fire_out(c, p)
      return 0

    lax.fori_loop(0, nloc, body, 0)
    pltpu.make_async_copy(tout.at[0], dst.at[pl.ds(0, 32), :], outsem).wait()
    pltpu.make_async_copy(tout.at[0], dst.at[pl.ds(0, 32), :], outsem).wait()

  do_table(enc_t, enc4)
  do_table(dec_t, dec4)


def _gather_body(in_hbm, ctx_hbm, neg_hbm, enc_hbm, dec_hbm,
                 pos_out, neg_out,
                 widx, cidx, nidx, wrows, crows, nrows, posbuf, negbuf, sem):
  wid = lax.axis_index("s") * NC + lax.axis_index("c")
  perm, pmask = _lane_helpers()

  def _combine(a, b, s):
    m = pmask[s]
    pm = perm(jnp.where(m, b, a), s)
    return jnp.where(m, a, pm) + jnp.where(m, pm, b)

  def chunk_body(c, _):
    base = pl.multiple_of((wid * NCH + c) * CB, CB)
    nbase = pl.multiple_of(base * NEG, NEG_CB)
    pltpu.sync_copy(in_hbm.at[pl.ds(base, CB)], widx)
    pltpu.sync_copy(ctx_hbm.at[pl.ds(base, CB)], cidx)
    pltpu.sync_copy(neg_hbm.at[pl.ds(nbase, NEG_CB)], nidx)

    handles = [
        pltpu.async_copy(enc_hbm.at[widx], wrows, sem),
        pltpu.async_copy(dec_hbm.at[cidx], crows, sem),
    ]
    for j in range(NEG_STREAMS):
      handles.append(
          pltpu.async_copy(dec_hbm.at[nidx.at[pl.ds(j * 128, 128)]],
                           nrows.at[pl.ds(j * 128, 128)], sem))
    for h in handles:
      h.wait()

    def b_body(i, _):
      w0 = wrows[i, pl.ds(0, LANES)]
      w1 = wrows[i, pl.ds(LANES, LANES)]
      c0 = crows[i, pl.ds(0, LANES)]
      c1 = crows[i, pl.ds(LANES, LANES)]
      p = w0 * c0 + w1 * c1
      for s in (8, 4, 2, 1):
        p = p + perm(p, s)
      posbuf[pl.ds(i, LANES)] = p
      r0 = i * NEG
      h = []
      for k in range(NEG):
        n0 = nrows[r0 + k, pl.ds(0, LANES)]
        n1 = nrows[r0 + k, pl.ds(LANES, LANES)]
        h.append(w0 * n0 + w1 * n1)
      vs = h[:LANES]
      for s in (8, 4, 2, 1):
        half = len(vs) // 2
        vs = [_combine(vs[j], vs[j + half], s) for j in range(half)]
      negbuf[pl.ds(r0, LANES)] = vs[0]
      rs = []
      for k in range(LANES, NEG):
        t = h[k]
        t = t + perm(t, 8)
        t = t + perm(t, 4)
        rs.append(t)
      rs = [_combine(rs[j], rs[j + 2], 2) for j in range(2)]
      rs = [_combine(rs[0], rs[1], 1)]
      negbuf[pl.ds(r0 + LANES, LANES)] = rs[0]
      return 0

    lax.fori_loop(0, CB, b_body, 0)

    pltpu.sync_copy(posbuf.at[pl.ds(0, CB)], pos_out.at[pl.ds(base, CB)])
    pltpu.sync_copy(negbuf.at[pl.ds(0, NEG_CB)],
                    neg_out.at[pl.ds(nbase, NEG_CB)])
    return 0

  lax.fori_loop(0, NCH, chunk_body, 0)


@jax.jit
def _w2v_call(in_flat, ctx_flat, neg_flat, enc_t, dec_t):
  mesh = plsc.VectorSubcoreMesh(core_axis_name="c", subcore_axis_name="s")
  conv = pl.kernel(
      _transpose_body,
      out_type=(
          jax.ShapeDtypeStruct((V4P, 128), jnp.float32),
          jax.ShapeDtypeStruct((V4P, 128), jnp.float32),
      ),
      mesh=mesh,
      scratch_types=[
          pltpu.VMEM((2, DIM, 128), jnp.float32),
          pltpu.VMEM((2, DIM, 128), jnp.float32),
          pltpu.SemaphoreType.DMA,
          pltpu.SemaphoreType.DMA,
      ],
      compiler_params=pltpu.CompilerParams(use_tc_tiling_on_sc=True),
  )
  enc4, dec4 = conv(enc_t, dec_t)
  encR = enc4.reshape(VP, DIM)
  decR = dec4.reshape(VP, DIM)
  kern = pl.kernel(
      _gather_body,
      out_type=(
          jax.ShapeDtypeStruct((B,), jnp.float32),
          jax.ShapeDtypeStruct((B * NEG,), jnp.float32),
      ),
      mesh=mesh,
      scratch_types=[
          pltpu.VMEM((CB,), jnp.int32),
          pltpu.VMEM((CB,), jnp.int32),
          pltpu.VMEM((NEG_CB,), jnp.int32),
          pltpu.VMEM((CB, DIM), jnp.float32),
          pltpu.VMEM((CB, DIM), jnp.float32),
          pltpu.VMEM((NEG_CB, DIM), jnp.float32),
          pltpu.VMEM((CB + LANES,), jnp.float32),
          pltpu.VMEM((NEG_CB + LANES,), jnp.float32),
          pltpu.SemaphoreType.DMA,
      ],
      compiler_params=pltpu.CompilerParams(use_tc_tiling_on_sc=False),
  )
  return kern(in_flat, ctx_flat, neg_flat, encR, decR)


def kernel(input_tokens, ctx_tokens, neg_tokens, encoder_weight, decoder_weight):
  in_flat = input_tokens.reshape(B).astype(jnp.int32)
  ctx_flat = ctx_tokens.reshape(B).astype(jnp.int32)
  neg_flat = neg_tokens.reshape(B * NEG).astype(jnp.int32)
  pos, neg = _w2v_call(in_flat, ctx_flat, neg_flat,
                       encoder_weight.T, decoder_weight.T)
  return pos.reshape(B, 1, 1), neg.reshape(B, 1, NEG)


# speedup vs baseline: 2.6300x; 1.1971x over previous
import jax
import jax.numpy as jnp
from jax import lax
from jax.experimental import pallas as pl
from jax.experimental.pallas import tpu as pltpu
from jax.experimental.pallas import tpu_sc as plsc

B = 16384
DIM = 32
NEG = 20
LANES = 16

VOCAB = 1000000

NC = 2
NS = 16
NW = NC * NS

NCHT = (VOCAB + 127) // 128
V4P = NCHT * 32
VP = V4P * 4
TCPW = (NCHT + NW - 1) // NW

B_PER_W = B // NW
CB = 128
NCH = B_PER_W // CB
NEG_CB = CB * NEG
NEG_STREAMS = NEG_CB // 128


def _lane_helpers():
  lane = lax.iota(jnp.int32, LANES)
  dn = lax.GatherDimensionNumbers(
      offset_dims=(), collapsed_slice_dims=(0,), start_index_map=(0,))
  pidx = {s: jnp.bitwise_xor(lane, s)[:, None] for s in (8, 4, 2, 1)}
  pmask = {s: (lane & s) == 0 for s in (8, 4, 2, 1)}

  def perm(v, s):
    return lax.gather(v, pidx[s], dn, (1,),
                      mode=lax.GatherScatterMode.PROMISE_IN_BOUNDS)

  return perm, pmask


def _transpose_body(enc_t, dec_t, in_hbm, dec4, wout,
                    tin, tout, widxv, wstage, wall, insem, outsem, wsem):
  wid = lax.axis_index("s") * NC + lax.axis_index("c")
  perm, pmask = _lane_helpers()

  def eklundh16(vs):
    for s in (8, 4, 2, 1):
      m = pmask[s]
      nv = list(vs)
      for i in range(LANES):
        if i & s:
          continue
        j = i ^ s
        a, b = vs[i], vs[j]
        pm = perm(jnp.where(m, b, a), s)
        nv[i] = jnp.where(m, a, pm)
        nv[j] = jnp.where(m, pm, b)
      vs = nv
    return vs

  def do_words():
    base = pl.multiple_of(wid * B_PER_W, B_PER_W)
    pltpu.sync_copy(in_hbm.at[pl.ds(base, B_PER_W)], widxv)
    lane = lax.iota(jnp.int32, LANES)
    dn = lax.GatherDimensionNumbers(
        offset_dims=(), collapsed_slice_dims=(0,), start_index_map=(0,))

    def group(g, _):
      g16 = pl.multiple_of(g * LANES, LANES)
      tv = widxv[pl.ds(g16, LANES)]
      handles = []
      for l in range(LANES):
        tb = pl.multiple_of(tv[l] & ~127, 128)
        handles.append(
            pltpu.async_copy(enc_t.at[:, pl.ds(tb, 128)],
                             wstage.at[l], wsem))
      for h in handles:
        h.wait()
      for l in range(LANES):
        colb = pl.multiple_of((tv[l] & 127) & ~15, LANES)
        cv = (jnp.zeros((LANES,), jnp.int32) + (tv[l] & 15))[:, None]
        for db in range(2):
          acc = None
          for r in range(LANES):
            vr = wstage[l, LANES * db + r, pl.ds(colb, LANES)]
            bc = lax.gather(vr, cv, dn, (1,),
                            mode=lax.GatherScatterMode.PROMISE_IN_BOUNDS)
            acc = bc if acc is None else jnp.where(lane == r, bc, acc)
          wall[g * 4 + (l >> 2),
               pl.ds((l & 3) * DIM + LANES * db, LANES)] = acc
      return 0

    lax.fori_loop(0, B_PER_W // LANES, group, 0)
    r0 = pl.multiple_of(wid * (B_PER_W * DIM // 128), 128)
    pltpu.sync_copy(wall, wout.at[pl.ds(r0, B_PER_W * DIM // 128), :])

  def do_table(src, dst):
    nloc = jnp.where(wid < NCHT - (TCPW - 1) * NW, TCPW, TCPW - 1)

    def fire_in(c, p):
      col0 = pl.multiple_of((wid + NW * c) * 128, 128)
      pltpu.async_copy(src.at[:, pl.ds(col0, 128)], tin.at[p], insem)

    def fire_out(c, p):
      r0 = pl.multiple_of((wid + NW * c) * 32, 32)
      pltpu.async_copy(tout.at[p], dst.at[pl.ds(r0, 32), :], outsem)

    fire_in(0, 0)

    def body(c, _):
      p = c & 1

      @pl.when(c + 1 < nloc)
      def _():
        fire_in(c + 1, 1 - p)

      pltpu.make_async_copy(src.at[:, pl.ds(0, 128)], tin.at[0], insem).wait()
      @pl.when(c >= 2)
      def _():
        pltpu.make_async_copy(tout.at[0], dst.at[pl.ds(0, 32), :],
                              outsem).wait()

      for db in range(2):
        for cb in range(8):
          vs = eklundh16(
              [tin[p, LANES * db + i, pl.ds(LANES * cb, LANES)]
               for i in range(LANES)])
          for j in range(LANES):
            tout[p, 4 * cb + (j >> 2),
                 pl.ds((j & 3) * DIM + LANES * db, LANES)] = vs[j]

      fire_out(c, p)
      return 0

    lax.fori_loop(0, nloc, body, 0)
    pltpu.make_async_copy(tout.at[0], dst.at[pl.ds(0, 32), :], outsem).wait()
    pltpu.make_async_copy(tout.at[0], dst.at[pl.ds(0, 32), :], outsem).wait()

  do_table(dec_t, dec4)
  do_words()


def _gather_body(in_hbm, ctx_hbm, neg_hbm, wview_hbm, dec_hbm,
                 pos_out, neg_out,
                 cidx, nidx, wrows, crows, nrows, posbuf, negbuf, sem):
  wid = lax.axis_index("s") * NC + lax.axis_index("c")
  perm, pmask = _lane_helpers()

  def _combine(a, b, s):
    m = pmask[s]
    pm = perm(jnp.where(m, b, a), s)
    return jnp.where(m, a, pm) + jnp.where(m, pm, b)

  def chunk_body(c, _):
    base = pl.multiple_of((wid * NCH + c) * CB, CB)
    nbase = pl.multiple_of(base * NEG, NEG_CB)
    pltpu.sync_copy(ctx_hbm.at[pl.ds(base, CB)], cidx)
    pltpu.sync_copy(neg_hbm.at[pl.ds(nbase, NEG_CB)], nidx)

    handles = [
        pltpu.async_copy(wview_hbm.at[pl.ds(base, CB), :], wrows, sem),
        pltpu.async_copy(dec_hbm.at[cidx], crows, sem),
    ]
    for j in range(NEG_STREAMS):
      handles.append(
          pltpu.async_copy(dec_hbm.at[nidx.at[pl.ds(j * 128, 128)]],
                           nrows.at[pl.ds(j * 128, 128)], sem))
    for h in handles:
      h.wait()

    def b_body(i, _):
      w0 = wrows[i, pl.ds(0, LANES)]
      w1 = wrows[i, pl.ds(LANES, LANES)]
      c0 = crows[i, pl.ds(0, LANES)]
      c1 = crows[i, pl.ds(LANES, LANES)]
      p = w0 * c0 + w1 * c1
      for s in (8, 4, 2, 1):
        p = p + perm(p, s)
      posbuf[pl.ds(i, LANES)] = p
      r0 = i * NEG
      h = []
      for k in range(NEG):
        n0 = nrows[r0 + k, pl.ds(0, LANES)]
        n1 = nrows[r0 + k, pl.ds(LANES, LANES)]
        h.append(w0 * n0 + w1 * n1)
      vs = h[:LANES]
      for s in (8, 4, 2, 1):
        half = len(vs) // 2
        vs = [_combine(vs[j], vs[j + half], s) for j in range(half)]
      negbuf[pl.ds(r0, LANES)] = vs[0]
      rs = []
      for k in range(LANES, NEG):
        t = h[k]
        t = t + perm(t, 8)
        t = t + perm(t, 4)
        rs.append(t)
      rs = [_combine(rs[j], rs[j + 2], 2) for j in range(2)]
      rs = [_combine(rs[0], rs[1], 1)]
      negbuf[pl.ds(r0 + LANES, LANES)] = rs[0]
      return 0

    lax.fori_loop(0, CB, b_body, 0)

    pltpu.sync_copy(posbuf.at[pl.ds(0, CB)], pos_out.at[pl.ds(base, CB)])
    pltpu.sync_copy(negbuf.at[pl.ds(0, NEG_CB)],
                    neg_out.at[pl.ds(nbase, NEG_CB)])
    return 0

  lax.fori_loop(0, NCH, chunk_body, 0)


@jax.jit
def _w2v_call(in_flat, ctx_flat, neg_flat, enc_t, dec_t):
  mesh = plsc.VectorSubcoreMesh(core_axis_name="c", subcore_axis_name="s")
  conv = pl.kernel(
      _transpose_body,
      out_type=(
          jax.ShapeDtypeStruct((V4P, 128), jnp.float32),
          jax.ShapeDtypeStruct((B * DIM // 128, 128), jnp.float32),
      ),
      mesh=mesh,
      scratch_types=[
          pltpu.VMEM((2, DIM, 128), jnp.float32),
          pltpu.VMEM((2, DIM, 128), jnp.float32),
          pltpu.VMEM((B_PER_W,), jnp.int32),
          pltpu.VMEM((LANES, DIM, 128), jnp.float32),
          pltpu.VMEM((B_PER_W * DIM // 128, 128), jnp.float32),
          pltpu.SemaphoreType.DMA,
          pltpu.SemaphoreType.DMA,
          pltpu.SemaphoreType.DMA,
      ],
      compiler_params=pltpu.CompilerParams(use_tc_tiling_on_sc=True),
  )
  dec4, wout = conv(enc_t, dec_t, in_flat)
  decR = dec4.reshape(VP, DIM)
  wviewR = wout.reshape(B, DIM)
  kern = pl.kernel(
      _gather_body,
      out_type=(
          jax.ShapeDtypeStruct((B,), jnp.float32),
          jax.ShapeDtypeStruct((B * NEG,), jnp.float32),
      ),
      mesh=mesh,
      scratch_types=[
          pltpu.VMEM((CB,), jnp.int32),
          pltpu.VMEM((NEG_CB,), jnp.int32),
          pltpu.VMEM((CB, DIM), jnp.float32),
          pltpu.VMEM((CB, DIM), jnp.float32),
          pltpu.VMEM((NEG_CB, DIM), jnp.float32),
          pltpu.VMEM((CB + LANES,), jnp.float32),
          pltpu.VMEM((NEG_CB + LANES,), jnp.float32),
          pltpu.SemaphoreType.DMA,
      ],
      compiler_params=pltpu.CompilerParams(use_tc_tiling_on_sc=False),
  )
  return kern(in_flat, ctx_flat, neg_flat, wviewR, decR)


def kernel(input_tokens, ctx_tokens, neg_tokens, encoder_weight, decoder_weight):
  in_flat = input_tokens.reshape(B).astype(jnp.int32)
  ctx_flat = ctx_tokens.reshape(B).astype(jnp.int32)
  neg_flat = neg_tokens.reshape(B * NEG).astype(jnp.int32)
  pos, neg = _w2v_call(in_flat, ctx_flat, neg_flat,
                       encoder_weight.T, decoder_weight.T)
  return pos.reshape(B, 1, 1), neg.reshape(B, 1, NEG)


# speedup vs baseline: 2.7143x; 1.0321x over previous
import jax
import jax.numpy as jnp
from jax import lax
from jax.experimental import pallas as pl
from jax.experimental.pallas import tpu as pltpu
from jax.experimental.pallas import tpu_sc as plsc

B = 16384
DIM = 32
NEG = 20
LANES = 16

VOCAB = 1000000

NC = 2
NS = 16
NW = NC * NS

NCHT = (VOCAB + 127) // 128
V4P = NCHT * 32
VP = V4P * 4
TCPW = (NCHT + NW - 1) // NW

B_PER_W = B // NW
CB = 64
NCH = B_PER_W // CB
NEG_CB = CB * NEG
NEG_STREAMS = NEG_CB // 128


def _lane_helpers():
  lane = lax.iota(jnp.int32, LANES)
  dn = lax.GatherDimensionNumbers(
      offset_dims=(), collapsed_slice_dims=(0,), start_index_map=(0,))
  pidx = {s: jnp.bitwise_xor(lane, s)[:, None] for s in (8, 4, 2, 1)}
  pmask = {s: (lane & s) == 0 for s in (8, 4, 2, 1)}

  def perm(v, s):
    return lax.gather(v, pidx[s], dn, (1,),
                      mode=lax.GatherScatterMode.PROMISE_IN_BOUNDS)

  return perm, pmask


def _transpose_body(enc_t, dec_t, in_hbm, dec4, wout,
                    tin, tout, widxv, wstage, wall, insem, outsem, wsem):
  wid = lax.axis_index("s") * NC + lax.axis_index("c")
  perm, pmask = _lane_helpers()

  def eklundh16(vs):
    for s in (8, 4, 2, 1):
      m = pmask[s]
      nv = list(vs)
      for i in range(LANES):
        if i & s:
          continue
        j = i ^ s
        a, b = vs[i], vs[j]
        pm = perm(jnp.where(m, b, a), s)
        nv[i] = jnp.where(m, a, pm)
        nv[j] = jnp.where(m, pm, b)
      vs = nv
    return vs

  def do_words():
    base = pl.multiple_of(wid * B_PER_W, B_PER_W)
    pltpu.sync_copy(in_hbm.at[pl.ds(base, B_PER_W)], widxv)
    lane = lax.iota(jnp.int32, LANES)
    dn = lax.GatherDimensionNumbers(
        offset_dims=(), collapsed_slice_dims=(0,), start_index_map=(0,))

    def group(g, _):
      g16 = pl.multiple_of(g * LANES, LANES)
      tv = widxv[pl.ds(g16, LANES)]
      handles = []
      for l in range(LANES):
        tb = pl.multiple_of(tv[l] & ~127, 128)
        handles.append(
            pltpu.async_copy(enc_t.at[:, pl.ds(tb, 128)],
                             wstage.at[l], wsem))
      for h in handles:
        h.wait()
      for l in range(LANES):
        colb = pl.multiple_of((tv[l] & 127) & ~15, LANES)
        cv = (jnp.zeros((LANES,), jnp.int32) + (tv[l] & 15))[:, None]
        for db in range(2):
          acc = None
          for r in range(LANES):
            vr = wstage[l, LANES * db + r, pl.ds(colb, LANES)]
            bc = lax.gather(vr, cv, dn, (1,),
                            mode=lax.GatherScatterMode.PROMISE_IN_BOUNDS)
            acc = bc if acc is None else jnp.where(lane == r, bc, acc)
          wall[g * 4 + (l >> 2),
               pl.ds((l & 3) * DIM + LANES * db, LANES)] = acc
      return 0

    lax.fori_loop(0, B_PER_W // LANES, group, 0)
    r0 = pl.multiple_of(wid * (B_PER_W * DIM // 128), 128)
    pltpu.sync_copy(wall, wout.at[pl.ds(r0, B_PER_W * DIM // 128), :])

  def do_table(src, dst):
    nloc = jnp.where(wid < NCHT - (TCPW - 1) * NW, TCPW, TCPW - 1)

    def fire_in(c, p):
      col0 = pl.multiple_of((wid + NW * c) * 128, 128)
      pltpu.async_copy(src.at[:, pl.ds(col0, 128)], tin.at[p], insem)

    def fire_out(c, p):
      r0 = pl.multiple_of((wid + NW * c) * 32, 32)
      pltpu.async_copy(tout.at[p], dst.at[pl.ds(r0, 32), :], outsem)

    fire_in(0, 0)

    def body(c, _):
      p = c & 1

      @pl.when(c + 1 < nloc)
      def _():
        fire_in(c + 1, 1 - p)

      pltpu.make_async_copy(src.at[:, pl.ds(0, 128)], tin.at[0], insem).wait()
      @pl.when(c >= 2)
      def _():
        pltpu.make_async_copy(tout.at[0], dst.at[pl.ds(0, 32), :],
                              outsem).wait()

      for db in range(2):
        for cb in range(8):
          vs = eklundh16(
              [tin[p, LANES * db + i, pl.ds(LANES * cb, LANES)]
               for i in range(LANES)])
          for j in range(LANES):
            tout[p, 4 * cb + (j >> 2),
                 pl.ds((j & 3) * DIM + LANES * db, LANES)] = vs[j]

      fire_out(c, p)
      return 0

    lax.fori_loop(0, nloc, body, 0)
    pltpu.make_async_copy(tout.at[0], dst.at[pl.ds(0, 32), :], outsem).wait()
    pltpu.make_async_copy(tout.at[0], dst.at[pl.ds(0, 32), :], outsem).wait()

  do_table(dec_t, dec4)
  do_words()


def _gather_body(in_hbm, ctx_hbm, neg_hbm, wview_hbm, dec_hbm,
                 pos_out, neg_out,
                 cidx, nidx, wrows, crows, nrows, posbuf, negbuf, sem):
  wid = lax.axis_index("s") * NC + lax.axis_index("c")
  perm, pmask = _lane_helpers()

  def _combine(a, b, s):
    m = pmask[s]
    pm = perm(jnp.where(m, b, a), s)
    return jnp.where(m, a, pm) + jnp.where(m, pm, b)

  def stage(c, p):
    base = pl.multiple_of((wid * NCH + c) * CB, CB)
    nbase = pl.multiple_of(base * NEG, NEG_CB)
    pltpu.sync_copy(ctx_hbm.at[pl.ds(base, CB)], cidx.at[p])
    pltpu.sync_copy(neg_hbm.at[pl.ds(nbase, NEG_CB)], nidx.at[p])
    pltpu.async_copy(wview_hbm.at[pl.ds(base, CB), :], wrows.at[p], sem)
    pltpu.async_copy(dec_hbm.at[cidx.at[p]], crows.at[p], sem)
    for j in range(NEG_STREAMS):
      pltpu.async_copy(dec_hbm.at[nidx.at[p, pl.ds(j * 128, 128)]],
                       nrows.at[p, pl.ds(j * 128, 128)], sem)

  stage(0, 0)

  def chunk_body(c, _):
    p = c & 1
    base = pl.multiple_of((wid * NCH + c) * CB, CB)
    nbase = pl.multiple_of(base * NEG, NEG_CB)
    pltpu.make_async_copy(wview_hbm.at[pl.ds(0, CB), :], wrows.at[0],
                          sem).wait()
    pltpu.make_async_copy(dec_hbm.at[cidx.at[0]], crows.at[0], sem).wait()
    for j in range(NEG_STREAMS):
      pltpu.make_async_copy(dec_hbm.at[nidx.at[0, pl.ds(0, 128)]],
                            nrows.at[0, pl.ds(0, 128)], sem).wait()

    @pl.when(c + 1 < NCH)
    def _():
      stage(c + 1, 1 - p)

    def b_body(i, _):
      w0 = wrows[p, i, pl.ds(0, LANES)]
      w1 = wrows[p, i, pl.ds(LANES, LANES)]
      c0 = crows[p, i, pl.ds(0, LANES)]
      c1 = crows[p, i, pl.ds(LANES, LANES)]
      pv = w0 * c0 + w1 * c1
      for s in (8, 4, 2, 1):
        pv = pv + perm(pv, s)
      posbuf[pl.ds(i, LANES)] = pv
      r0 = i * NEG
      h = []
      for k in range(NEG):
        n0 = nrows[p, r0 + k, pl.ds(0, LANES)]
        n1 = nrows[p, r0 + k, pl.ds(LANES, LANES)]
        h.append(w0 * n0 + w1 * n1)
      vs = h[:LANES]
      for s in (8, 4, 2, 1):
        half = len(vs) // 2
        vs = [_combine(vs[j], vs[j + half], s) for j in range(half)]
      negbuf[pl.ds(r0, LANES)] = vs[0]
      rs = []
      for k in range(LANES, NEG):
        t = h[k]
        t = t + perm(t, 8)
        t = t + perm(t, 4)
        rs.append(t)
      rs = [_combine(rs[j], rs[j + 2], 2) for j in range(2)]
      rs = [_combine(rs[0], rs[1], 1)]
      negbuf[pl.ds(r0 + LANES, LANES)] = rs[0]
      return 0

    lax.fori_loop(0, CB, b_body, 0)

    pltpu.sync_copy(posbuf.at[pl.ds(0, CB)], pos_out.at[pl.ds(base, CB)])
    pltpu.sync_copy(negbuf.at[pl.ds(0, NEG_CB)],
                    neg_out.at[pl.ds(nbase, NEG_CB)])
    return 0

  lax.fori_loop(0, NCH, chunk_body, 0)


@jax.jit
def _w2v_call(in_flat, ctx_flat, neg_flat, enc_t, dec_t):
  mesh = plsc.VectorSubcoreMesh(core_axis_name="c", subcore_axis_name="s")
  conv = pl.kernel(
      _transpose_body,
      out_type=(
          jax.ShapeDtypeStruct((V4P, 128), jnp.float32),
          jax.ShapeDtypeStruct((B * DIM // 128, 128), jnp.float32),
      ),
      mesh=mesh,
      scratch_types=[
          pltpu.VMEM((2, DIM, 128), jnp.float32),
          pltpu.VMEM((2, DIM, 128), jnp.float32),
          pltpu.VMEM((B_PER_W,), jnp.int32),
          pltpu.VMEM((LANES, DIM, 128), jnp.float32),
          pltpu.VMEM((B_PER_W * DIM // 128, 128), jnp.float32),
          pltpu.SemaphoreType.DMA,
          pltpu.SemaphoreType.DMA,
          pltpu.SemaphoreType.DMA,
      ],
      compiler_params=pltpu.CompilerParams(use_tc_tiling_on_sc=True),
  )
  dec4, wout = conv(enc_t, dec_t, in_flat)
  decR = dec4.reshape(VP, DIM)
  wviewR = wout.reshape(B, DIM)
  kern = pl.kernel(
      _gather_body,
      out_type=(
          jax.ShapeDtypeStruct((B,), jnp.float32),
          jax.ShapeDtypeStruct((B * NEG,), jnp.float32),
      ),
      mesh=mesh,
      scratch_types=[
          pltpu.VMEM((2, CB), jnp.int32),
          pltpu.VMEM((2, NEG_CB), jnp.int32),
          pltpu.VMEM((2, CB, DIM), jnp.float32),
          pltpu.VMEM((2, CB, DIM), jnp.float32),
          pltpu.VMEM((2, NEG_CB, DIM), jnp.float32),
          pltpu.VMEM((CB + LANES,), jnp.float32),
          pltpu.VMEM((NEG_CB + LANES,), jnp.float32),
          pltpu.SemaphoreType.DMA,
      ],
      compiler_params=pltpu.CompilerParams(use_tc_tiling_on_sc=False),
  )
  return kern(in_flat, ctx_flat, neg_flat, wviewR, decR)


def kernel(input_tokens, ctx_tokens, neg_tokens, encoder_weight, decoder_weight):
  in_flat = input_tokens.reshape(B).astype(jnp.int32)
  ctx_flat = ctx_tokens.reshape(B).astype(jnp.int32)
  neg_flat = neg_tokens.reshape(B * NEG).astype(jnp.int32)
  pos, neg = _w2v_call(in_flat, ctx_flat, neg_flat,
                       encoder_weight.T, decoder_weight.T)
  return pos.reshape(B, 1, 1), neg.reshape(B, 1, NEG)


# speedup vs baseline: 3.1278x; 1.1523x over previous
import jax
import jax.numpy as jnp
from jax import lax
from jax.experimental import pallas as pl
from jax.experimental.pallas import tpu as pltpu
from jax.experimental.pallas import tpu_sc as plsc

B = 16384
DIM = 32
NEG = 20
LANES = 16

VOCAB = 1000000

NC = 2
NS = 16
NW = NC * NS

NCHT = (VOCAB + 127) // 128
V4P = NCHT * 32
VP = V4P * 4
TCPW = (NCHT + NW - 1) // NW

B_PER_W = B // NW
WG = 8
NWG = B_PER_W // WG
CB = 64
NCH = B_PER_W // CB
NEG_CB = CB * NEG
NEG_STREAMS = NEG_CB // 128


def _lane_helpers():
  lane = lax.iota(jnp.int32, LANES)
  dn = lax.GatherDimensionNumbers(
      offset_dims=(), collapsed_slice_dims=(0,), start_index_map=(0,))
  pidx = {s: jnp.bitwise_xor(lane, s)[:, None] for s in (8, 4, 2, 1)}
  pmask = {s: (lane & s) == 0 for s in (8, 4, 2, 1)}

  def perm(v, s):
    return lax.gather(v, pidx[s], dn, (1,),
                      mode=lax.GatherScatterMode.PROMISE_IN_BOUNDS)

  return perm, pmask


def _transpose_body(enc_t, dec_t, in_hbm, dec4, wout,
                    tin, tout, widxv, wstage, wall, insem, outsem, wsem):
  wid = lax.axis_index("s") * NC + lax.axis_index("c")
  perm, pmask = _lane_helpers()

  def eklundh16(vs):
    for s in (8, 4, 2, 1):
      m = pmask[s]
      nv = list(vs)
      for i in range(LANES):
        if i & s:
          continue
        j = i ^ s
        a, b = vs[i], vs[j]
        pm = perm(jnp.where(m, b, a), s)
        nv[i] = jnp.where(m, a, pm)
        nv[j] = jnp.where(m, pm, b)
      vs = nv
    return vs

  lane = lax.iota(jnp.int32, LANES)
  dn = lax.GatherDimensionNumbers(
      offset_dims=(), collapsed_slice_dims=(0,), start_index_map=(0,))

  def word_fire(g, hn):
    tvg = widxv[pl.ds(g * WG, LANES)]
    for l in range(WG):
      tb = pl.multiple_of(tvg[l] & ~127, 128)
      pltpu.async_copy(enc_t.at[:, pl.ds(tb, 128)], wstage.at[hn, l], wsem)

  def word_step(g):
    h = g & 1
    for _l in range(WG):
      pltpu.make_async_copy(enc_t.at[:, pl.ds(0, 128)], wstage.at[0, 0],
                            wsem).wait()

    @pl.when(g + 1 < NWG)
    def _():
      word_fire(g + 1, 1 - (g & 1))

    tv = widxv[pl.ds(g * WG, LANES)]
    for l in range(WG):
      colb = pl.multiple_of((tv[l] & 127) & ~15, LANES)
      cv = (jnp.zeros((LANES,), jnp.int32) + (tv[l] & 15))[:, None]
      for db in range(2):
        acc = None
        for r in range(LANES):
          vr = wstage[h, l, LANES * db + r, pl.ds(colb, LANES)]
          bc = lax.gather(vr, cv, dn, (1,),
                          mode=lax.GatherScatterMode.PROMISE_IN_BOUNDS)
          acc = bc if acc is None else jnp.where(lane == r, bc, acc)
        wall[g * (WG // 4) + (l >> 2),
             pl.ds((l & 3) * DIM + LANES * db, LANES)] = acc

  def do_table(src, dst, words=False):
    nloc = jnp.where(wid < NCHT - (TCPW - 1) * NW, TCPW, TCPW - 1)

    def fire_in(c, p):
      col0 = pl.multiple_of((wid + NW * c) * 128, 128)
      pltpu.async_copy(src.at[:, pl.ds(col0, 128)], tin.at[p], insem)

    def fire_out(c, p):
      r0 = pl.multiple_of((wid + NW * c) * 32, 32)
      pltpu.async_copy(tout.at[p], dst.at[pl.ds(r0, 32), :], outsem)

    fire_in(0, 0)

    def body(c, _):
      p = c & 1

      @pl.when(c + 1 < nloc)
      def _():
        fire_in(c + 1, 1 - p)

      pltpu.make_async_copy(src.at[:, pl.ds(0, 128)], tin.at[0], insem).wait()
      @pl.when(c >= 2)
      def _():
        pltpu.make_async_copy(tout.at[0], dst.at[pl.ds(0, 32), :],
                              outsem).wait()

      for db in range(2):
        for cb in range(8):
          vs = eklundh16(
              [tin[p, LANES * db + i, pl.ds(LANES * cb, LANES)]
               for i in range(LANES)])
          for j in range(LANES):
            tout[p, 4 * cb + (j >> 2),
                 pl.ds((j & 3) * DIM + LANES * db, LANES)] = vs[j]

      fire_out(c, p)
      if words:
        @pl.when(((c & 3) == 0) & ((c >> 2) < NWG - 3))
        def _():
          word_step(c >> 2)
      return 0

    lax.fori_loop(0, nloc, body, 0)
    pltpu.make_async_copy(tout.at[0], dst.at[pl.ds(0, 32), :], outsem).wait()
    pltpu.make_async_copy(tout.at[0], dst.at[pl.ds(0, 32), :], outsem).wait()

  base = pl.multiple_of(wid * B_PER_W, B_PER_W)
  pltpu.sync_copy(in_hbm.at[pl.ds(base, B_PER_W)],
                  widxv.at[pl.ds(0, B_PER_W)])
  word_fire(0, 0)
  do_table(dec_t, dec4, words=True)
  for g in range(NWG - 3, NWG):
    word_step(g)
  r0 = pl.multiple_of(wid * (B_PER_W * DIM // 128), 128)
  pltpu.sync_copy(wall, wout.at[pl.ds(r0, B_PER_W * DIM // 128), :])


def _gather_body(in_hbm, ctx_hbm, neg_hbm, wview_hbm, dec_hbm,
                 pos_out, neg_out,
                 cidx, nidx, wrows, crows, nrows, posbuf, negbuf, sem):
  wid = lax.axis_index("s") * NC + lax.axis_index("c")
  perm, pmask = _lane_helpers()

  def _combine(a, b, s):
    m = pmask[s]
    pm = perm(jnp.where(m, b, a), s)
    return jnp.where(m, a, pm) + jnp.where(m, pm, b)

  def stage(c, p):
    base = pl.multiple_of((wid * NCH + c) * CB, CB)
    nbase = pl.multiple_of(base * NEG, NEG_CB)
    pltpu.sync_copy(ctx_hbm.at[pl.ds(base, CB)], cidx.at[p])
    pltpu.sync_copy(neg_hbm.at[pl.ds(nbase, NEG_CB)], nidx.at[p])
    pltpu.async_copy(wview_hbm.at[pl.ds(base, CB), :], wrows.at[p], sem)
    pltpu.async_copy(dec_hbm.at[cidx.at[p]], crows.at[p], sem)
    for j in range(NEG_STREAMS):
      pltpu.async_copy(dec_hbm.at[nidx.at[p, pl.ds(j * 128, 128)]],
                       nrows.at[p, pl.ds(j * 128, 128)], sem)

  stage(0, 0)

  def chunk_body(c, _):
    p = c & 1
    base = pl.multiple_of((wid * NCH + c) * CB, CB)
    nbase = pl.multiple_of(base * NEG, NEG_CB)
    pltpu.make_async_copy(wview_hbm.at[pl.ds(0, CB), :], wrows.at[0],
                          sem).wait()
    pltpu.make_async_copy(dec_hbm.at[cidx.at[0]], crows.at[0], sem).wait()
    for j in range(NEG_STREAMS):
      pltpu.make_async_copy(dec_hbm.at[nidx.at[0, pl.ds(0, 128)]],
                            nrows.at[0, pl.ds(0, 128)], sem).wait()

    @pl.when(c + 1 < NCH)
    def _():
      stage(c + 1, 1 - p)

    def b_body(i, _):
      w0 = wrows[p, i, pl.ds(0, LANES)]
      w1 = wrows[p, i, pl.ds(LANES, LANES)]
      c0 = crows[p, i, pl.ds(0, LANES)]
      c1 = crows[p, i, pl.ds(LANES, LANES)]
      pv = w0 * c0 + w1 * c1
      for s in (8, 4, 2, 1):
        pv = pv + perm(pv, s)
      posbuf[pl.ds(i, LANES)] = pv
      r0 = i * NEG
      h = []
      for k in range(NEG):
        n0 = nrows[p, r0 + k, pl.ds(0, LANES)]
        n1 = nrows[p, r0 + k, pl.ds(LANES, LANES)]
        h.append(w0 * n0 + w1 * n1)
      vs = h[:LANES]
      for s in (8, 4, 2, 1):
        half = len(vs) // 2
        vs = [_combine(vs[j], vs[j + half], s) for j in range(half)]
      negbuf[pl.ds(r0, LANES)] = vs[0]
      rs = []
      for k in range(LANES, NEG):
        t = h[k]
        t = t + perm(t, 8)
        t = t + perm(t, 4)
        rs.append(t)
      rs = [_combine(rs[j], rs[j + 2], 2) for j in range(2)]
      rs = [_combine(rs[0], rs[1], 1)]
      negbuf[pl.ds(r0 + LANES, LANES)] = rs[0]
      return 0

    lax.fori_loop(0, CB, b_body, 0)

    pltpu.sync_copy(posbuf.at[pl.ds(0, CB)], pos_out.at[pl.ds(base, CB)])
    pltpu.sync_copy(negbuf.at[pl.ds(0, NEG_CB)],
                    neg_out.at[pl.ds(nbase, NEG_CB)])
    return 0

  lax.fori_loop(0, NCH, chunk_body, 0)


@jax.jit
def _w2v_call(in_flat, ctx_flat, neg_flat, enc_t, dec_t):
  mesh = plsc.VectorSubcoreMesh(core_axis_name="c", subcore_axis_name="s")
  conv = pl.kernel(
      _transpose_body,
      out_type=(
          jax.ShapeDtypeStruct((V4P, 128), jnp.float32),
          jax.ShapeDtypeStruct((B * DIM // 128, 128), jnp.float32),
      ),
      mesh=mesh,
      scratch_types=[
          pltpu.VMEM((2, DIM, 128), jnp.float32),
          pltpu.VMEM((2, DIM, 128), jnp.float32),
          pltpu.VMEM((B_PER_W + LANES,), jnp.int32),
          pltpu.VMEM((2, WG, DIM, 128), jnp.float32),
          pltpu.VMEM((B_PER_W * DIM // 128, 128), jnp.float32),
          pltpu.SemaphoreType.DMA,
          pltpu.SemaphoreType.DMA,
          pltpu.SemaphoreType.DMA,
      ],
      compiler_params=pltpu.CompilerParams(use_tc_tiling_on_sc=True),
  )
  dec4, wout = conv(enc_t, dec_t, in_flat)
  decR = dec4.reshape(VP, DIM)
  wviewR = wout.reshape(B, DIM)
  kern = pl.kernel(
      _gather_body,
      out_type=(
          jax.ShapeDtypeStruct((B,), jnp.float32),
          jax.ShapeDtypeStruct((B * NEG,), jnp.float32),
      ),
      mesh=mesh,
      scratch_types=[
          pltpu.VMEM((2, CB), jnp.int32),
          pltpu.VMEM((2, NEG_CB), jnp.int32),
          pltpu.VMEM((2, CB, DIM), jnp.float32),
          pltpu.VMEM((2, CB, DIM), jnp.float32),
          pltpu.VMEM((2, NEG_CB, DIM), jnp.float32),
          pltpu.VMEM((CB + LANES,), jnp.float32),
          pltpu.VMEM((NEG_CB + LANES,), jnp.float32),
          pltpu.SemaphoreType.DMA,
      ],
      compiler_params=pltpu.CompilerParams(use_tc_tiling_on_sc=False),
  )
  return kern(in_flat, ctx_flat, neg_flat, wviewR, decR)


def kernel(input_tokens, ctx_tokens, neg_tokens, encoder_weight, decoder_weight):
  in_flat = input_tokens.reshape(B).astype(jnp.int32)
  ctx_flat = ctx_tokens.reshape(B).astype(jnp.int32)
  neg_flat = neg_tokens.reshape(B * NEG).astype(jnp.int32)
  pos, neg = _w2v_call(in_flat, ctx_flat, neg_flat,
                       encoder_weight.T, decoder_weight.T)
  return pos.reshape(B, 1, 1), neg.reshape(B, 1, NEG)
